# baseline (speedup 1.0000x reference)
import functools

import jax
import jax.numpy as jnp
import numpy as np
from jax import lax
from jax.experimental import pallas as pl
from jax.experimental.pallas import tpu as pltpu
from jax.experimental.pallas import tpu_sc as plsc

D = 128
SEQ = 128
LANES = 16
NVEC = D // LANES
OFF = (0, 32, 64, 96, 16, 48, 80, 112)


def _worker_count():
  info = plsc.get_sparse_core_info()
  return info.num_cores, info.num_subcores


@functools.lru_cache(maxsize=None)
def _build(n_events):
  nc, ns = _worker_count()
  nw = nc * ns
  assert n_events % nw == 0
  ev_w = n_events // nw

  mesh = plsc.VectorSubcoreMesh(core_axis_name="c", subcore_axis_name="s")

  @functools.partial(
      pl.kernel,
      mesh=mesh,
      name="event_encoder",
      compiler_params=pltpu.CompilerParams(
          needs_layout_passes=False, use_tc_tiling_on_sc=False),
      out_type=jax.ShapeDtypeStruct((nw, ev_w, D), jnp.float32),
      scratch_types=[
          pltpu.VMEM((ev_w, SEQ), jnp.int32),
          pltpu.VMEM((ev_w, SEQ), jnp.int32),
          pltpu.VMEM((ev_w, SEQ), jnp.int32),
          pltpu.VMEM((2 * SEQ, D), jnp.float32),
          pltpu.VMEM((2 * 2 * SEQ, D // 2), jnp.uint32),
          pltpu.VMEM((ev_w, D), jnp.float32),
          pltpu.SemaphoreType.DMA,
          pltpu.SemaphoreType.DMA,
      ],
  )
  def encoder(ii_hbm, ti_hbm, di_hbm, tab_i, tab_t, tab_d, out_hbm,
              idx_i, idx_t, idx_d, rows_f, rows_u, out_buf, sem0, sem1):
    wid = lax.axis_index("s") * nc + lax.axis_index("c")

    pltpu.sync_copy(ii_hbm.at[wid], idx_i)
    pltpu.sync_copy(ti_hbm.at[wid], idx_t)
    pltpu.sync_copy(di_hbm.at[wid], idx_d)

    def copies(e, slot, sem):
      return (
          pltpu.make_async_copy(
              tab_i.at[idx_i.at[e]], rows_f.at[pl.ds(slot * SEQ, SEQ)], sem),
          pltpu.make_async_copy(
              tab_t.at[idx_t.at[e]],
              rows_u.at[pl.ds(slot * 2 * SEQ, SEQ)], sem),
          pltpu.make_async_copy(
              tab_d.at[idx_d.at[e]],
              rows_u.at[pl.ds(slot * 2 * SEQ + SEQ, SEQ)], sem),
      )

    def issue(e, slot, sem):
      for c in copies(e, slot, sem):
        c.start()

    def wait(e, slot, sem):
      for c in copies(e, slot, sem):
        c.wait()

    def reduce_into(e, slot):
      def red_f(r, accs):
        return tuple(a + rows_f[slot * SEQ + r, pl.ds(j * LANES, LANES)]
                     for j, a in enumerate(accs))

      accs_f = lax.fori_loop(
          0, SEQ, red_f,
          tuple(jnp.zeros((LANES,), jnp.float32) for _ in range(NVEC)),
          unroll=16)

      def red_u(r, accs):
        new = list(accs)
        for c in range(NVEC // 2):
          w = rows_u[slot * 2 * SEQ + r, pl.ds(c * LANES, LANES)]
          a = plsc.bitcast(w << 16, jnp.float32)
          b = plsc.bitcast(w, jnp.float32)
          new[2 * c] = new[2 * c] + a
          new[2 * c + 1] = new[2 * c + 1] + b
        return tuple(new)

      accs_u = lax.fori_loop(
          0, 2 * SEQ, red_u,
          tuple(jnp.zeros((LANES,), jnp.float32) for _ in range(NVEC)),
          unroll=16)

      scale = jnp.float32(1.0 / SEQ)
      for j in range(NVEC):
        out_buf[e, pl.ds(OFF[j], LANES)] = (
            accs_f[j] + accs_u[OFF[j] // LANES]) * scale

    assert ev_w % 2 == 0
    issue(0, 0, sem0)

    def pair_body(k, carry):
      e0 = 2 * k
      issue(e0 + 1, 1, sem1)
      wait(e0, 0, sem0)
      reduce_into(e0, 0)

      @pl.when(e0 + 2 < ev_w)
      def _():
        issue(e0 + 2, 0, sem0)

      wait(e0 + 1, 1, sem1)
      reduce_into(e0 + 1, 1)
      return carry

    lax.fori_loop(0, ev_w // 2, pair_body, 0)
    pltpu.sync_copy(out_buf, out_hbm.at[wid])

  return encoder


def _to_packed_u32(table):
  bits = jax.lax.bitcast_convert_type(table, jnp.uint32)
  rnd = (bits + jnp.uint32(0x7FFF) + ((bits >> 16) & jnp.uint32(1))) >> 16
  return rnd[:, :D // 2] | (rnd[:, D // 2:] << 16)


def kernel(input_idx, type_idx, dpe_idx, E_input, E_type, E_dpe):
  b, l, seq = input_idx.shape
  n = b * l
  nc, ns = _worker_count()
  nw = nc * ns
  out = _build(n)(
      input_idx.reshape(nw, n // nw, seq).astype(jnp.int32),
      type_idx.reshape(nw, n // nw, seq).astype(jnp.int32),
      dpe_idx.reshape(nw, n // nw, seq).astype(jnp.int32),
      E_input,
      _to_packed_u32(E_type),
      _to_packed_u32(E_dpe),
  )
  perm = np.concatenate(
      [np.arange(16) + 32 * c for c in range(4)]
      + [np.arange(16) + 32 * c + 16 for c in range(4)])
  out = out[:, :, perm]
  return out.reshape(b, l, D)


# speedup vs baseline: 1.0114x; 1.0114x over previous
import functools

import jax
import jax.numpy as jnp
import numpy as np
from jax import lax
from jax.experimental import pallas as pl
from jax.experimental.pallas import tpu as pltpu
from jax.experimental.pallas import tpu_sc as plsc

D = 128
SEQ = 128
LANES = 16
NVEC = D // LANES
OFF = (0, 32, 64, 96, 16, 48, 80, 112)


def _worker_count():
  info = plsc.get_sparse_core_info()
  return info.num_cores, info.num_subcores


@functools.lru_cache(maxsize=None)
def _build(n_events):
  nc, ns = _worker_count()
  nw = nc * ns
  assert n_events % nw == 0
  ev_w = n_events // nw

  mesh = plsc.VectorSubcoreMesh(core_axis_name="c", subcore_axis_name="s")

  @functools.partial(
      pl.kernel,
      mesh=mesh,
      name="event_encoder",
      compiler_params=pltpu.CompilerParams(
          needs_layout_passes=False, use_tc_tiling_on_sc=False),
      out_type=jax.ShapeDtypeStruct((nw, ev_w, D), jnp.float32),
      scratch_types=[
          pltpu.VMEM((ev_w, SEQ), jnp.int32),
          pltpu.VMEM((ev_w, SEQ), jnp.int32),
          pltpu.VMEM((ev_w, SEQ), jnp.int32),
          pltpu.VMEM((2 * SEQ, D), jnp.float32),
          pltpu.VMEM((2 * 2 * SEQ, D // 2), jnp.uint32),
          pltpu.VMEM((ev_w, D), jnp.float32),
          pltpu.SemaphoreType.DMA,
          pltpu.SemaphoreType.DMA,
      ],
  )
  def encoder(ii_hbm, ti_hbm, di_hbm, tab_i, tab_t, tab_d, out_hbm,
              idx_i, idx_t, idx_d, rows_f, rows_u, out_buf, sem0, sem1):
    wid = lax.axis_index("s") * nc + lax.axis_index("c")

    pltpu.sync_copy(ii_hbm.at[wid], idx_i)
    pltpu.sync_copy(ti_hbm.at[wid], idx_t)
    pltpu.sync_copy(di_hbm.at[wid], idx_d)

    def copies(e, slot, sem):
      return (
          pltpu.make_async_copy(
              tab_i.at[idx_i.at[e]], rows_f.at[pl.ds(slot * SEQ, SEQ)], sem),
          pltpu.make_async_copy(
              tab_t.at[idx_t.at[e]],
              rows_u.at[pl.ds(slot * 2 * SEQ, SEQ)], sem),
          pltpu.make_async_copy(
              tab_d.at[idx_d.at[e]],
              rows_u.at[pl.ds(slot * 2 * SEQ + SEQ, SEQ)], sem),
      )

    def issue(e, slot, sem):
      for c in copies(e, slot, sem):
        c.start()

    def wait(e, slot, sem):
      for c in copies(e, slot, sem):
        c.wait()

    def reduce_into(e, slot):
      def red_f(r, accs):
        return tuple(a + rows_f[slot * SEQ + r, pl.ds(j * LANES, LANES)]
                     for j, a in enumerate(accs))

      accs_f = lax.fori_loop(
          0, SEQ, red_f,
          tuple(jnp.zeros((LANES,), jnp.float32) for _ in range(NVEC)),
          unroll=8)

      def red_u(r, accs):
        new = list(accs)
        for c in range(NVEC // 2):
          w = rows_u[slot * 2 * SEQ + r, pl.ds(c * LANES, LANES)]
          a = plsc.bitcast(w << 16, jnp.float32)
          b = plsc.bitcast(w, jnp.float32)
          new[2 * c] = new[2 * c] + a
          new[2 * c + 1] = new[2 * c + 1] + b
        return tuple(new)

      accs_u = lax.fori_loop(
          0, 2 * SEQ, red_u,
          tuple(jnp.zeros((LANES,), jnp.float32) for _ in range(NVEC)),
          unroll=8)

      scale = jnp.float32(1.0 / SEQ)
      for j in range(NVEC):
        out_buf[e, pl.ds(OFF[j], LANES)] = (
            accs_f[j] + accs_u[OFF[j] // LANES]) * scale

    assert ev_w % 2 == 0
    issue(0, 0, sem0)

    def pair_body(k, carry):
      e0 = 2 * k
      issue(e0 + 1, 1, sem1)
      wait(e0, 0, sem0)
      reduce_into(e0, 0)

      @pl.when(e0 + 2 < ev_w)
      def _():
        issue(e0 + 2, 0, sem0)

      wait(e0 + 1, 1, sem1)
      reduce_into(e0 + 1, 1)
      return carry

    lax.fori_loop(0, ev_w // 2, pair_body, 0)
    pltpu.sync_copy(out_buf, out_hbm.at[wid])

  return encoder


def _to_packed_u32(table):
  bits = jax.lax.bitcast_convert_type(table, jnp.uint32)
  rnd = (bits + jnp.uint32(0x7FFF) + ((bits >> 16) & jnp.uint32(1))) >> 16
  return rnd[:, :D // 2] | (rnd[:, D // 2:] << 16)


def kernel(input_idx, type_idx, dpe_idx, E_input, E_type, E_dpe):
  b, l, seq = input_idx.shape
  n = b * l
  nc, ns = _worker_count()
  nw = nc * ns
  out = _build(n)(
      input_idx.reshape(nw, n // nw, seq).astype(jnp.int32),
      type_idx.reshape(nw, n // nw, seq).astype(jnp.int32),
      dpe_idx.reshape(nw, n // nw, seq).astype(jnp.int32),
      E_input,
      _to_packed_u32(E_type),
      _to_packed_u32(E_dpe),
  )
  perm = np.concatenate(
      [np.arange(16) + 32 * c for c in range(4)]
      + [np.arange(16) + 32 * c + 16 for c in range(4)])
  out = out[:, :, perm]
  return out.reshape(b, l, D)
